# baseline (speedup 1.0000x reference)
import functools

import jax
import jax.numpy as jnp
from jax import lax
from jax.experimental import pallas as pl
from jax.experimental.pallas import tpu as pltpu
from jax.experimental.pallas import tpu_sc as plsc

_VOCAB = 100000
_DIM = 32
_BATCH = 1024
_CTX = 4

_NC = 2
_NS = 16
_NW = _NC * _NS
_ROWS_PER_W = _BATCH // _NW
_IDX_PER_W = _ROWS_PER_W * _CTX
_LANES = 16


def _emb_body(table_hbm, idx_hbm, out_hbm, idx_v, rows_v, emb_v, sem):
    wid = lax.axis_index("s") * _NC + lax.axis_index("c")
    pltpu.sync_copy(idx_hbm.at[pl.ds(wid * _IDX_PER_W, _IDX_PER_W)], idx_v)
    pltpu.async_copy(table_hbm.at[idx_v], rows_v, sem).wait()
    for i in range(_ROWS_PER_W):
        for h in range(_DIM // _LANES):
            c = pl.ds(h * _LANES, _LANES)
            emb_v[i, c] = (rows_v[_CTX * i, c] + rows_v[_CTX * i + 1, c]
                           + rows_v[_CTX * i + 2, c] + rows_v[_CTX * i + 3, c])
    pltpu.sync_copy(emb_v, out_hbm.at[pl.ds(wid * _ROWS_PER_W, _ROWS_PER_W)])


@functools.partial(jax.jit, static_argnames=())
def _sc_embed_sum(emb_table, idx_flat):
    mesh = plsc.VectorSubcoreMesh(core_axis_name="c", subcore_axis_name="s")
    k = functools.partial(
        pl.kernel,
        mesh=mesh,
        out_type=jax.ShapeDtypeStruct((_BATCH, _DIM), jnp.float32),
        scratch_types=[
            pltpu.VMEM((_IDX_PER_W,), jnp.int32),
            pltpu.VMEM((_IDX_PER_W, _DIM), jnp.float32),
            pltpu.VMEM((_ROWS_PER_W, _DIM), jnp.float32),
            pltpu.SemaphoreType.DMA,
        ],
        compiler_params=pltpu.CompilerParams(use_tc_tiling_on_sc=False),
    )(_emb_body)
    return k(emb_table, idx_flat)


_VC = 1024


def _mm_body(emb_ref, w_ref, b_ref, out_ref):
    out_ref[...] = lax.dot_general(
        emb_ref[...], w_ref[...],
        dimension_numbers=(((1,), (1,)), ((), ())),
        preferred_element_type=jnp.float32,
    ) + b_ref[...]


def _tc_project(emb, W, b2d):
    n_tiles = pl.cdiv(_VOCAB, _VC)
    return pl.pallas_call(
        _mm_body,
        grid=(n_tiles,),
        in_specs=[
            pl.BlockSpec((_BATCH, _DIM), lambda i: (0, 0)),
            pl.BlockSpec((_VC, _DIM), lambda i: (i, 0)),
            pl.BlockSpec((1, _VC), lambda i: (0, i)),
        ],
        out_specs=pl.BlockSpec((_BATCH, _VC), lambda i: (0, i)),
        out_shape=jax.ShapeDtypeStruct((_BATCH, _VOCAB), jnp.float32),
    )(emb, W, b2d)


def kernel(inputs, emb_table, W, b):
    idx_flat = inputs.reshape(-1).astype(jnp.int32)
    emb = _sc_embed_sum(emb_table, idx_flat)
    return _tc_project(emb, W, b.reshape(1, _VOCAB))


# speedup vs baseline: 1.9453x; 1.9453x over previous
import functools

import jax
import jax.numpy as jnp
from jax import lax
from jax.experimental import pallas as pl
from jax.experimental.pallas import tpu as pltpu
from jax.experimental.pallas import tpu_sc as plsc

_VOCAB = 100000
_DIM = 32
_BATCH = 1024
_CTX = 4

_NC = 2
_NS = 16
_NW = _NC * _NS
_ROWS_PER_W = _BATCH // _NW
_IDX_PER_W = _ROWS_PER_W * _CTX
_LANES = 16


def _emb_body(table_hbm, idx_hbm, out_hbm, idx_v, rows_v, emb_v, sem):
    wid = lax.axis_index("s") * _NC + lax.axis_index("c")
    pltpu.sync_copy(idx_hbm.at[pl.ds(wid * _IDX_PER_W, _IDX_PER_W)], idx_v)
    pltpu.async_copy(table_hbm.at[idx_v], rows_v, sem).wait()
    for i in range(_ROWS_PER_W):
        for h in range(_DIM // _LANES):
            c = pl.ds(h * _LANES, _LANES)
            emb_v[i, c] = (rows_v[_CTX * i, c] + rows_v[_CTX * i + 1, c]
                           + rows_v[_CTX * i + 2, c] + rows_v[_CTX * i + 3, c])
    pltpu.sync_copy(emb_v, out_hbm.at[pl.ds(wid * _ROWS_PER_W, _ROWS_PER_W)])


@functools.partial(jax.jit, static_argnames=())
def _sc_embed_sum(emb_table, idx_flat):
    mesh = plsc.VectorSubcoreMesh(core_axis_name="c", subcore_axis_name="s")
    k = functools.partial(
        pl.kernel,
        mesh=mesh,
        out_type=jax.ShapeDtypeStruct((_BATCH, _DIM), jnp.float32),
        scratch_types=[
            pltpu.VMEM((_IDX_PER_W,), jnp.int32),
            pltpu.VMEM((_IDX_PER_W, _DIM), jnp.float32),
            pltpu.VMEM((_ROWS_PER_W, _DIM), jnp.float32),
            pltpu.SemaphoreType.DMA,
        ],
        compiler_params=pltpu.CompilerParams(use_tc_tiling_on_sc=False),
    )(_emb_body)
    return k(emb_table, idx_flat)


_VC = 1024


def _mm_body(w_ref, emb_ref, b_ref, out_ref):
    out_ref[...] = lax.dot_general(
        w_ref[...], emb_ref[...],
        dimension_numbers=(((1,), (1,)), ((), ())),
        preferred_element_type=jnp.float32,
    ) + b_ref[...]


def _tc_project_t(emb, W, b2d):
    n_tiles = pl.cdiv(_VOCAB, _VC)
    return pl.pallas_call(
        _mm_body,
        grid=(n_tiles,),
        in_specs=[
            pl.BlockSpec((_VC, _DIM), lambda i: (i, 0)),
            pl.BlockSpec((_BATCH, _DIM), lambda i: (0, 0)),
            pl.BlockSpec((_VC, 1), lambda i: (i, 0)),
        ],
        out_specs=pl.BlockSpec((_VC, _BATCH), lambda i: (i, 0)),
        out_shape=jax.ShapeDtypeStruct((_VOCAB, _BATCH), jnp.float32),
    )(W, emb, b2d)


def kernel(inputs, emb_table, W, b):
    idx_flat = inputs.reshape(-1).astype(jnp.int32)
    emb = _sc_embed_sum(emb_table, idx_flat)
    out_t = _tc_project_t(emb, W, b.reshape(_VOCAB, 1))
    return out_t.T


# speedup vs baseline: 3.4753x; 1.7865x over previous
import functools

import jax
import jax.numpy as jnp
from jax import lax
from jax.experimental import pallas as pl
from jax.experimental.pallas import tpu as pltpu
from jax.experimental.pallas import tpu_sc as plsc

_VOCAB = 100000
_DIM = 32
_BATCH = 1024
_CTX = 4

_NC = 2
_NS = 16
_NW = _NC * _NS
_LANES = 16
_NCHUNK = _BATCH // _LANES


def _emb_body(tab_t_hbm, idx_hbm, out_hbm, row_v, idx_v, emb_v, sem):
    d = lax.axis_index("s") * _NC + lax.axis_index("c")
    pltpu.sync_copy(tab_t_hbm.at[d], row_v)
    pltpu.sync_copy(idx_hbm.at[:], idx_v)
    for c in range(_NCHUNK):
        acc = plsc.load_gather(row_v, [idx_v[pl.ds(16 * c, _LANES)]])
        for j in range(1, _CTX):
            acc = acc + plsc.load_gather(
                row_v, [idx_v[pl.ds(j * _BATCH + 16 * c, _LANES)]])
        emb_v[pl.ds(16 * c, _LANES)] = acc
    pltpu.sync_copy(emb_v, out_hbm.at[d])


def _sc_embed_sum(tab_t, idx_t):
    mesh = plsc.VectorSubcoreMesh(core_axis_name="c", subcore_axis_name="s")
    k = functools.partial(
        pl.kernel,
        mesh=mesh,
        out_type=jax.ShapeDtypeStruct((_DIM, _BATCH), jnp.float32),
        scratch_types=[
            pltpu.VMEM((_VOCAB,), jnp.float32),
            pltpu.VMEM((_CTX * _BATCH,), jnp.int32),
            pltpu.VMEM((_BATCH,), jnp.float32),
            pltpu.SemaphoreType.DMA,
        ],
        compiler_params=pltpu.CompilerParams(
            use_tc_tiling_on_sc=True, needs_layout_passes=False),
    )(_emb_body)
    return k(tab_t, idx_t)


_VC = 1024


def _mm_body(wt_ref, emb_t_ref, b_ref, out_ref):
    acc = lax.dot_general(
        wt_ref[...], emb_t_ref[...],
        dimension_numbers=(((0,), (0,)), ((), ())),
        preferred_element_type=jnp.float32,
    )
    out_ref[...] = acc + b_ref[...].reshape(_VC, 1)


def _tc_project_t(emb_t, Wt, b):
    n_tiles = pl.cdiv(_VOCAB, _VC)
    return pl.pallas_call(
        _mm_body,
        grid=(n_tiles,),
        in_specs=[
            pl.BlockSpec((_DIM, _VC), lambda i: (0, i)),
            pl.BlockSpec((_DIM, _BATCH), lambda i: (0, 0)),
            pl.BlockSpec((_VC,), lambda i: (i,)),
        ],
        out_specs=pl.BlockSpec((_VC, _BATCH), lambda i: (i, 0)),
        out_shape=jax.ShapeDtypeStruct((_VOCAB, _BATCH), jnp.float32),
    )(Wt, emb_t, b)


def kernel(inputs, emb_table, W, b):
    idx_t = inputs.T.reshape(-1).astype(jnp.int32)
    emb_t = _sc_embed_sum(emb_table.T, idx_t)
    out_t = _tc_project_t(emb_t, W.T, b)
    return out_t.T


# speedup vs baseline: 3.9443x; 1.1349x over previous
import functools

import jax
import jax.numpy as jnp
from jax import lax
from jax.experimental import pallas as pl
from jax.experimental.pallas import tpu as pltpu
from jax.experimental.pallas import tpu_sc as plsc

_VOCAB = 100000
_DIM = 32
_BATCH = 1024
_CTX = 4

_NC = 2
_NS = 16
_NW = _NC * _NS
_LANES = 16
_NCHUNK = _BATCH // _LANES


def _emb_body(tab_t_hbm, idx_hbm, out_hbm, row_v, idx_v, emb_v, sem):
    d = lax.axis_index("s") * _NC + lax.axis_index("c")
    pltpu.sync_copy(tab_t_hbm.at[d], row_v)
    pltpu.sync_copy(idx_hbm.at[:], idx_v)
    for c in range(_NCHUNK):
        acc = plsc.load_gather(row_v, [idx_v[pl.ds(16 * c, _LANES)]])
        for j in range(1, _CTX):
            acc = acc + plsc.load_gather(
                row_v, [idx_v[pl.ds(j * _BATCH + 16 * c, _LANES)]])
        emb_v[pl.ds(16 * c, _LANES)] = acc
    pltpu.sync_copy(emb_v, out_hbm.at[d])


def _sc_embed_sum(tab_t, idx_t):
    mesh = plsc.VectorSubcoreMesh(core_axis_name="c", subcore_axis_name="s")
    k = functools.partial(
        pl.kernel,
        mesh=mesh,
        out_type=jax.ShapeDtypeStruct((_DIM, _BATCH), jnp.float32),
        scratch_types=[
            pltpu.VMEM((_VOCAB,), jnp.float32),
            pltpu.VMEM((_CTX * _BATCH,), jnp.int32),
            pltpu.VMEM((_BATCH,), jnp.float32),
            pltpu.SemaphoreType.DMA,
        ],
        compiler_params=pltpu.CompilerParams(
            use_tc_tiling_on_sc=True, needs_layout_passes=False),
    )(_emb_body)
    return k(tab_t, idx_t)


_VC = 2048


def _mm_body(wt_ref, emb_t_ref, b_ref, out_ref):
    acc = lax.dot_general(
        wt_ref[...], emb_t_ref[...],
        dimension_numbers=(((0,), (0,)), ((), ())),
        preferred_element_type=jnp.float32,
    )
    out_ref[...] = acc + b_ref[...].reshape(_VC, 1)


def _tc_project_t(emb_t, Wt, b):
    n_tiles = pl.cdiv(_VOCAB, _VC)
    return pl.pallas_call(
        _mm_body,
        grid=(n_tiles,),
        in_specs=[
            pl.BlockSpec((_DIM, _VC), lambda i: (0, i)),
            pl.BlockSpec((_DIM, _BATCH), lambda i: (0, 0)),
            pl.BlockSpec((_VC,), lambda i: (i,)),
        ],
        out_specs=pl.BlockSpec((_VC, _BATCH), lambda i: (i, 0)),
        out_shape=jax.ShapeDtypeStruct((_VOCAB, _BATCH), jnp.float32),
    )(Wt, emb_t, b)


def kernel(inputs, emb_table, W, b):
    idx_t = inputs.T.reshape(-1).astype(jnp.int32)
    emb_t = _sc_embed_sum(emb_table.T, idx_t)
    out_t = _tc_project_t(emb_t, W.T, b)
    return out_t.T


# speedup vs baseline: 3.9545x; 1.0026x over previous
import functools

import jax
import jax.numpy as jnp
from jax import lax
from jax.experimental import pallas as pl
from jax.experimental.pallas import tpu as pltpu
from jax.experimental.pallas import tpu_sc as plsc

_VOCAB = 100000
_DIM = 32
_BATCH = 1024
_CTX = 4

_NC = 2
_NS = 16
_NW = _NC * _NS
_LANES = 16
_NCHUNK = _BATCH // _LANES


def _emb_body(tab_t_hbm, idx_hbm, out_hbm, row_v, idx_v, emb_v, sem):
    d = lax.axis_index("s") * _NC + lax.axis_index("c")
    pltpu.sync_copy(tab_t_hbm.at[d], row_v)
    pltpu.sync_copy(idx_hbm.at[:], idx_v)
    for c in range(_NCHUNK):
        acc = plsc.load_gather(row_v, [idx_v[pl.ds(16 * c, _LANES)]])
        for j in range(1, _CTX):
            acc = acc + plsc.load_gather(
                row_v, [idx_v[pl.ds(j * _BATCH + 16 * c, _LANES)]])
        emb_v[pl.ds(16 * c, _LANES)] = acc
    pltpu.sync_copy(emb_v, out_hbm.at[d])


def _sc_embed_sum(tab_t, idx_t):
    mesh = plsc.VectorSubcoreMesh(core_axis_name="c", subcore_axis_name="s")
    k = functools.partial(
        pl.kernel,
        mesh=mesh,
        out_type=jax.ShapeDtypeStruct((_DIM, _BATCH), jnp.float32),
        scratch_types=[
            pltpu.VMEM((_VOCAB,), jnp.float32),
            pltpu.VMEM((_CTX * _BATCH,), jnp.int32),
            pltpu.VMEM((_BATCH,), jnp.float32),
            pltpu.SemaphoreType.DMA,
        ],
        compiler_params=pltpu.CompilerParams(
            use_tc_tiling_on_sc=True, needs_layout_passes=False),
    )(_emb_body)
    return k(tab_t, idx_t)


_VC = 4096


def _mm_body(wt_ref, emb_t_ref, b_ref, out_ref):
    acc = lax.dot_general(
        wt_ref[...], emb_t_ref[...],
        dimension_numbers=(((0,), (0,)), ((), ())),
        preferred_element_type=jnp.float32,
    )
    out_ref[...] = acc + b_ref[...].reshape(_VC, 1)


def _tc_project_t(emb_t, Wt, b):
    n_tiles = pl.cdiv(_VOCAB, _VC)
    return pl.pallas_call(
        _mm_body,
        grid=(n_tiles,),
        in_specs=[
            pl.BlockSpec((_DIM, _VC), lambda i: (0, i)),
            pl.BlockSpec((_DIM, _BATCH), lambda i: (0, 0)),
            pl.BlockSpec((_VC,), lambda i: (i,)),
        ],
        out_specs=pl.BlockSpec((_VC, _BATCH), lambda i: (i, 0)),
        out_shape=jax.ShapeDtypeStruct((_VOCAB, _BATCH), jnp.float32),
    )(Wt, emb_t, b)


def kernel(inputs, emb_table, W, b):
    idx_t = inputs.T.reshape(-1).astype(jnp.int32)
    emb_t = _sc_embed_sum(emb_table.T, idx_t)
    out_t = _tc_project_t(emb_t, W.T, b)
    return out_t.T
